# baseline (speedup 1.0000x reference)
import functools

import jax
import jax.numpy as jnp
from jax import lax
from jax.experimental import pallas as pl
from jax.experimental.pallas import tpu as pltpu
from jax.experimental.pallas import tpu_sc as plsc

F32 = jnp.float32
BF16 = jnp.bfloat16
I32 = jnp.int32
NC = 2
NS = 16
L = 16

EBLK = 2000


def _bf16_bits(x):
    return lax.bitcast_convert_type(x.astype(BF16), jnp.uint16).astype(I32)


def _pq_body(x_ref, wp_ref, wq_ref, p_ref, q_ref):
    x = x_ref[...]
    h = x.shape[1] // 2

    def packc(m):
        return lax.shift_left(_bf16_bits(m[:, h:]), 16) | _bf16_bits(m[:, :h])
    p_ref[...] = packc(jnp.dot(x, wp_ref[...], preferred_element_type=F32))
    q_ref[...] = packc(jnp.dot(x, wq_ref[...], preferred_element_type=F32))


def _edge_body(ef_ref, we_ref, b_ref, e_ref):
    ef = ef_ref[...]
    for t in range(4):
        e0 = jnp.dot(ef[:, 32 * t:32 * t + 16], we_ref[...],
                     preferred_element_type=F32) + b_ref[...]
        e1 = jnp.dot(ef[:, 32 * t + 16:32 * t + 32], we_ref[...],
                     preferred_element_type=F32) + b_ref[...]
        e_ref[pl.ds(EBLK * t, EBLK), :] = \
            lax.shift_left(_bf16_bits(e1), 16) | _bf16_bits(e0)


def _post_body(x_ref, hp_ref, dp_ref, w2_ref, b2_ref, wua_ref, wub_ref,
               bu1_ref, wu2_ref, bu2_ref, g_ref, be_ref, o_ref):
    x = x_ref[...]
    hsum = hp_ref[0] + hp_ref[1]
    deg = dp_ref[0][:, 0:1] + dp_ref[1][:, 0:1]
    agg = jnp.dot(hsum, w2_ref[...], preferred_element_type=F32) \
        + deg * b2_ref[...]
    u1 = jnp.maximum(
        jnp.dot(x, wua_ref[...], preferred_element_type=F32)
        + jnp.dot(agg, wub_ref[...], preferred_element_type=F32)
        + bu1_ref[...], 0.0)
    upd = jnp.dot(u1, wu2_ref[...], preferred_element_type=F32) + bu2_ref[...]
    y = x + upd
    mu = jnp.mean(y, axis=-1, keepdims=True)
    var = jnp.mean((y - mu) ** 2, axis=-1, keepdims=True)
    o_ref[...] = (y - mu) * lax.rsqrt(var + 1e-5) * g_ref[...] + be_ref[...]


def _make_sc_edge_kernel(n_nodes, n_edges, hid):
    nw = NC * NS
    npk = n_edges // 2
    rows_pt = npk // nw
    B = 20
    KC = 50
    n_ss = rows_pt // (KC * B)
    pairs = KC // 2
    rpt = n_nodes // NS
    B2 = 2 * B
    hw = hid // 2
    mask_hi = jnp.int32(-65536)

    def body(p_hbm, q_hbm, e_hbm, dst_hbm, src_hbm,
             h_out, d_out,
             h_sh, d_sh,
             dbuf, sbuf,
             pr0, qr0, er0, pr1, qr1, er1,
             hr0, hr1,
             ones_v, zd,
             sp0, sq0, se0, sp1, sq1, se1,
             shh0, shd0, shh1, shd1):
        cid = lax.axis_index("c")
        sid = lax.axis_index("s")
        wid = cid * NS + sid

        zero16 = jnp.zeros((L,), F32)
        ones16 = jnp.ones((L,), F32)

        def fill(i, c):
            for j in range(hid // L):
                hr0[i, pl.ds(j * L, L)] = zero16
            zd[i, pl.ds(0, L)] = zero16
            ones_v[i, pl.ds(0, L)] = ones16
            return c
        lax.fori_loop(0, B2, fill, 0)

        nfull = rpt // B2
        rem = rpt - nfull * B2
        for t in range(nfull):
            pltpu.sync_copy(hr0, h_sh.at[pl.ds(sid * rpt + t * B2, B2)])
            pltpu.sync_copy(zd, d_sh.at[pl.ds(sid * rpt + t * B2, B2)])
        if rem:
            pltpu.sync_copy(hr0.at[pl.ds(0, rem)],
                            h_sh.at[pl.ds(sid * rpt + nfull * B2, rem)])
            pltpu.sync_copy(zd.at[pl.ds(0, rem)],
                            d_sh.at[pl.ds(sid * rpt + nfull * B2, rem)])
        plsc.subcore_barrier()

        sets = ((pr0, qr0, er0, hr0, sp0, sq0, se0, shh0, shd0),
                (pr1, qr1, er1, hr1, sp1, sq1, se1, shh1, shd1))

        def issue_gather(row_base, c, bset):
            pr, qr, er, hr, sp, sq, se, shh, shd = bset
            pltpu.async_copy(p_hbm.at[dbuf.at[c]], pr, sp)
            pltpu.async_copy(q_hbm.at[sbuf.at[c]], qr, sq)
            pltpu.async_copy(e_hbm.at[pl.ds(row_base + c * B, B)], er, se)

        def wait_gather(bset):
            pr, qr, er, hr, sp, sq, se, shh, shd = bset
            pltpu.make_async_copy(p_hbm.at[pl.ds(0, B2)], pr, sp).wait()
            pltpu.make_async_copy(q_hbm.at[pl.ds(0, B2)], qr, sq).wait()
            pltpu.make_async_copy(e_hbm.at[pl.ds(0, B)], er, se).wait()

        def issue_scatter(c, bset):
            pr, qr, er, hr, sp, sq, se, shh, shd = bset
            pltpu.async_copy(hr, h_sh.at[dbuf.at[c]], shh, add=True)
            pltpu.async_copy(ones_v, d_sh.at[dbuf.at[c]], shd, add=True)

        def wait_scatter(bset):
            pr, qr, er, hr, sp, sq, se, shh, shd = bset
            pltpu.make_async_copy(hr, h_sh.at[pl.ds(0, B2)], shh).wait()
            pltpu.make_async_copy(ones_v, d_sh.at[pl.ds(0, B2)], shd).wait()

        def split(w):
            lo = lax.bitcast_convert_type(lax.shift_left(w, 16), F32)
            hi = lax.bitcast_convert_type(lax.bitwise_and(w, mask_hi), F32)
            return lo, hi

        def compute(bset):
            pr, qr, er, hr, sp, sq, se, shh, shd = bset

            def row(rr, cc):
                for t in range(4):
                    sl = pl.ds(t * L, L)
                    sh = pl.ds(hw + t * L, L)
                    pll, plh = split(pr[rr, sl])
                    phl, phh = split(pr[B + rr, sl])
                    qll, qlh = split(qr[rr, sl])
                    qhl, qhh = split(qr[B + rr, sl])
                    eal, eah = split(er[rr, sl])
                    ebl, ebh = split(er[rr, sh])
                    hr[rr, sl] = jnp.maximum(pll + qll + eal, 0.0)
                    hr[rr, sh] = jnp.maximum(plh + qlh + ebl, 0.0)
                    hr[B + rr, sl] = jnp.maximum(phl + qhl + eah, 0.0)
                    hr[B + rr, sh] = jnp.maximum(phh + qhh + ebh, 0.0)
                return cc
            lax.fori_loop(0, B, row, 0)

        def superstep(s, c):
            row_base = wid * rows_pt + s * KC * B
            idx_row = wid * (rows_pt // B) + s * KC
            pltpu.sync_copy(dst_hbm.at[pl.ds(idx_row, KC)], dbuf)
            pltpu.sync_copy(src_hbm.at[pl.ds(idx_row, KC)], sbuf)
            issue_gather(row_base, 0, sets[0])
            issue_gather(row_base, 1, sets[1])

            def pair(i, cc):
                for b in (0, 1):
                    ch = 2 * i + b
                    wait_gather(sets[b])

                    @pl.when(i > 0)
                    def _():
                        wait_scatter(sets[b])
                    compute(sets[b])
                    issue_scatter(ch, sets[b])

                    @pl.when(i < pairs - 1)
                    def _():
                        issue_gather(row_base, ch + 2, sets[b])
                return cc
            lax.fori_loop(0, pairs, pair, 0)
            wait_scatter(sets[0])
            wait_scatter(sets[1])
            return c
        lax.fori_loop(0, n_ss, superstep, 0)

        plsc.subcore_barrier()
        pltpu.sync_copy(h_sh.at[pl.ds(sid * rpt, rpt)],
                        h_out.at[cid, pl.ds(sid * rpt, rpt)])
        pltpu.sync_copy(d_sh.at[pl.ds(sid * rpt, rpt)],
                        d_out.at[cid, pl.ds(sid * rpt, rpt)])

    mesh = plsc.VectorSubcoreMesh(core_axis_name="c", subcore_axis_name="s")
    return pl.kernel(
        body,
        out_type=[
            jax.ShapeDtypeStruct((NC, n_nodes, hid), F32),
            jax.ShapeDtypeStruct((NC, n_nodes, L), F32),
        ],
        mesh=mesh,
        compiler_params=pltpu.CompilerParams(use_tc_tiling_on_sc=False),
        scratch_types=[
            pltpu.VMEM_SHARED((n_nodes, hid), F32),
            pltpu.VMEM_SHARED((n_nodes, L), F32),
            pltpu.VMEM((KC, B2), I32),
            pltpu.VMEM((KC, B2), I32),
            pltpu.VMEM((B2, hw), I32),
            pltpu.VMEM((B2, hw), I32),
            pltpu.VMEM((B, hid), I32),
            pltpu.VMEM((B2, hw), I32),
            pltpu.VMEM((B2, hw), I32),
            pltpu.VMEM((B, hid), I32),
            pltpu.VMEM((B2, hid), F32),
            pltpu.VMEM((B2, hid), F32),
            pltpu.VMEM((B2, L), F32),
            pltpu.VMEM((B2, L), F32),
            pltpu.SemaphoreType.DMA,
            pltpu.SemaphoreType.DMA,
            pltpu.SemaphoreType.DMA,
            pltpu.SemaphoreType.DMA,
            pltpu.SemaphoreType.DMA,
            pltpu.SemaphoreType.DMA,
            pltpu.SemaphoreType.DMA,
            pltpu.SemaphoreType.DMA,
            pltpu.SemaphoreType.DMA,
            pltpu.SemaphoreType.DMA,
        ],
    )


def kernel(node_features, edge_index, edge_features, mW1, mb1, mW2, mb2,
           uW1, ub1, uW2, ub2, gamma, beta):
    x = node_features
    n, hid = x.shape
    ne, edim = edge_features.shape
    src_flat = edge_index[0].astype(I32)
    dst_flat = edge_index[1].astype(I32)

    npk = ne // 2
    nblk = ne // (8 * EBLK)

    def remap(v):
        return v.reshape(nblk, EBLK // 20, 20, 4, 2).transpose(
            (0, 3, 1, 4, 2)).reshape(npk // 20, 40)
    dstx = remap(dst_flat)
    srcx = remap(src_flat)

    wp = mW1[:hid]
    wq = mW1[hid:2 * hid]
    we = mW1[2 * hid:].astype(BF16)

    nb = 10
    bn = n // nb
    p, q = pl.pallas_call(
        _pq_body,
        grid=(nb,),
        in_specs=[
            pl.BlockSpec((bn, hid), lambda i: (i, 0)),
            pl.BlockSpec((hid, hid), lambda i: (0, 0)),
            pl.BlockSpec((hid, hid), lambda i: (0, 0)),
        ],
        out_specs=[
            pl.BlockSpec((bn, hid // 2), lambda i: (i, 0)),
            pl.BlockSpec((bn, hid // 2), lambda i: (i, 0)),
        ],
        out_shape=[
            jax.ShapeDtypeStruct((n, hid // 2), I32),
            jax.ShapeDtypeStruct((n, hid // 2), I32),
        ],
    )(x, wp, wq)

    ef2 = edge_features.astype(BF16).reshape(ne // 8, 128)
    e = pl.pallas_call(
        _edge_body,
        grid=(nblk,),
        in_specs=[
            pl.BlockSpec((EBLK, 128), lambda i: (i, 0)),
            pl.BlockSpec((edim, hid), lambda i: (0, 0)),
            pl.BlockSpec((1, hid), lambda i: (0, 0)),
        ],
        out_specs=pl.BlockSpec((4 * EBLK, hid), lambda i: (i, 0)),
        out_shape=jax.ShapeDtypeStruct((npk, hid), I32),
    )(ef2, we, mb1.reshape(1, hid))

    hpart, dpart = _make_sc_edge_kernel(n, ne, hid)(p, q, e, dstx, srcx)

    out = pl.pallas_call(
        _post_body,
        grid=(nb,),
        in_specs=[
            pl.BlockSpec((bn, hid), lambda i: (i, 0)),
            pl.BlockSpec((NC, bn, hid), lambda i: (0, i, 0)),
            pl.BlockSpec((NC, bn, L), lambda i: (0, i, 0)),
            pl.BlockSpec((hid, hid), lambda i: (0, 0)),
            pl.BlockSpec((1, hid), lambda i: (0, 0)),
            pl.BlockSpec((hid, hid), lambda i: (0, 0)),
            pl.BlockSpec((hid, hid), lambda i: (0, 0)),
            pl.BlockSpec((1, hid), lambda i: (0, 0)),
            pl.BlockSpec((hid, hid), lambda i: (0, 0)),
            pl.BlockSpec((1, hid), lambda i: (0, 0)),
            pl.BlockSpec((1, hid), lambda i: (0, 0)),
            pl.BlockSpec((1, hid), lambda i: (0, 0)),
        ],
        out_specs=pl.BlockSpec((bn, hid), lambda i: (i, 0)),
        out_shape=jax.ShapeDtypeStruct((n, hid), F32),
    )(x, hpart, dpart, mW2, mb2.reshape(1, hid), uW1[:hid], uW1[hid:],
      ub1.reshape(1, hid), uW2, ub2.reshape(1, hid), gamma.reshape(1, hid),
      beta.reshape(1, hid))
    return out


# speedup vs baseline: 1.5872x; 1.5872x over previous
import functools

import jax
import jax.numpy as jnp
from jax import lax
from jax.experimental import pallas as pl
from jax.experimental.pallas import tpu as pltpu
from jax.experimental.pallas import tpu_sc as plsc

F32 = jnp.float32
BF16 = jnp.bfloat16
I32 = jnp.int32
NC = 2
NS = 16
L = 16

EBLK = 8000


def _bf16_bits(x):
    return lax.bitcast_convert_type(x.astype(BF16), jnp.uint16).astype(I32)


def _pq_body(x_ref, wp_ref, wq_ref, p_ref, q_ref):
    x = x_ref[...]
    h = x.shape[1] // 2

    def packc(m):
        return lax.shift_left(_bf16_bits(m[:, h:]), 16) | _bf16_bits(m[:, :h])
    p_ref[...] = packc(jnp.dot(x, wp_ref[...], preferred_element_type=F32))
    q_ref[...] = packc(jnp.dot(x, wq_ref[...], preferred_element_type=F32))


def _edge_body(ef_ref, we_ref, b_ref, e_ref):
    half = EBLK // 2
    ea = jnp.dot(ef_ref[...].astype(BF16), we_ref[...],
                 preferred_element_type=F32) + b_ref[...]
    e_ref[...] = lax.shift_left(_bf16_bits(ea[half:]), 16) \
        | _bf16_bits(ea[:half])


def _post_body(x_ref, hp_ref, w2_ref, b2_ref, wua_ref, wub_ref,
               bu1_ref, wu2_ref, bu2_ref, g_ref, be_ref, o_ref):
    x = x_ref[...]
    hall = hp_ref[0] + hp_ref[1]
    hsum = hall[:, :128]
    deg = hall[:, 128:129]
    agg = jnp.dot(hsum, w2_ref[...], preferred_element_type=F32) \
        + deg * b2_ref[...]
    u1 = jnp.maximum(
        jnp.dot(x, wua_ref[...], preferred_element_type=F32)
        + jnp.dot(agg, wub_ref[...], preferred_element_type=F32)
        + bu1_ref[...], 0.0)
    upd = jnp.dot(u1, wu2_ref[...], preferred_element_type=F32) + bu2_ref[...]
    y = x + upd
    mu = jnp.mean(y, axis=-1, keepdims=True)
    var = jnp.mean((y - mu) ** 2, axis=-1, keepdims=True)
    o_ref[...] = (y - mu) * lax.rsqrt(var + 1e-5) * g_ref[...] + be_ref[...]


def _make_sc_edge_kernel(n_nodes, n_edges, hid):
    nw = NC * NS
    npk = n_edges // 2
    rows_pt = npk // nw
    B = 20
    KC = 50
    n_ss = rows_pt // (KC * B)
    pairs = KC // 2
    rpt = n_nodes // NS
    B2 = 2 * B
    hw = hid // 2
    mask_hi = jnp.int32(-65536)

    hidw = hid + L

    def body(p_hbm, q_hbm, e_hbm, dst_hbm, src_hbm,
             h_out,
             h_sh,
             dbuf, sbuf,
             pr0, qr0, er0, pr1, qr1, er1,
             hr0, hr1,
             sp0, sq0, se0, sp1, sq1, se1,
             shh0, shh1):
        cid = lax.axis_index("c")
        sid = lax.axis_index("s")
        wid = cid * NS + sid

        zero16 = jnp.zeros((L,), F32)
        ones16 = jnp.ones((L,), F32)

        def fill(i, c):
            for j in range(hidw // L):
                hr0[i, pl.ds(j * L, L)] = zero16
                hr1[i, pl.ds(j * L, L)] = zero16
            return c
        lax.fori_loop(0, B2, fill, 0)

        nfull = rpt // B2
        rem = rpt - nfull * B2
        for t in range(nfull):
            pltpu.async_copy(hr0, h_sh.at[pl.ds(sid * rpt + t * B2, B2)],
                             sp0)
        if rem:
            pltpu.async_copy(hr0.at[pl.ds(0, rem)],
                             h_sh.at[pl.ds(sid * rpt + nfull * B2, rem)],
                             sp0)
        for t in range(nfull):
            pltpu.make_async_copy(
                hr0, h_sh.at[pl.ds(0, B2)], sp0).wait()
        if rem:
            pltpu.make_async_copy(
                hr0.at[pl.ds(0, rem)], h_sh.at[pl.ds(0, rem)], sp0).wait()

        def fillones(i, c):
            hr0[i, pl.ds(hid, L)] = ones16
            hr1[i, pl.ds(hid, L)] = ones16
            return c
        lax.fori_loop(0, B2, fillones, 0)
        plsc.subcore_barrier()

        sets = ((pr0, qr0, er0, hr0, sp0, sq0, se0, shh0),
                (pr1, qr1, er1, hr1, sp1, sq1, se1, shh1))

        def issue_gather(row_base, c, bset):
            pr, qr, er, hr, sp, sq, se, shh = bset
            pltpu.async_copy(p_hbm.at[dbuf.at[c]], pr, sp)
            pltpu.async_copy(q_hbm.at[sbuf.at[c]], qr, sq)
            pltpu.async_copy(e_hbm.at[pl.ds(row_base + c * B, B)], er, se)

        def wait_gather(bset):
            pr, qr, er, hr, sp, sq, se, shh = bset
            pltpu.make_async_copy(p_hbm.at[pl.ds(0, B2)], pr, sp).wait()
            pltpu.make_async_copy(q_hbm.at[pl.ds(0, B2)], qr, sq).wait()
            pltpu.make_async_copy(e_hbm.at[pl.ds(0, B)], er, se).wait()

        def issue_scatter(c, bset):
            pr, qr, er, hr, sp, sq, se, shh = bset
            pltpu.async_copy(hr, h_sh.at[dbuf.at[c]], shh, add=True)

        def wait_scatter(bset):
            pr, qr, er, hr, sp, sq, se, shh = bset
            pltpu.make_async_copy(hr, h_sh.at[pl.ds(0, B2)], shh).wait()

        def split(w):
            lo = lax.bitcast_convert_type(lax.shift_left(w, 16), F32)
            hi = lax.bitcast_convert_type(lax.bitwise_and(w, mask_hi), F32)
            return lo, hi

        def compute(bset):
            pr, qr, er, hr, sp, sq, se, shh = bset

            def row(rr, cc):
                for t in range(4):
                    sl = pl.ds(t * L, L)
                    sh = pl.ds(hw + t * L, L)
                    pll, plh = split(pr[rr, sl])
                    phl, phh = split(pr[B + rr, sl])
                    qll, qlh = split(qr[rr, sl])
                    qhl, qhh = split(qr[B + rr, sl])
                    eal, eah = split(er[rr, sl])
                    ebl, ebh = split(er[rr, sh])
                    hr[rr, sl] = jnp.maximum(pll + qll + eal, 0.0)
                    hr[rr, sh] = jnp.maximum(plh + qlh + ebl, 0.0)
                    hr[B + rr, sl] = jnp.maximum(phl + qhl + eah, 0.0)
                    hr[B + rr, sh] = jnp.maximum(phh + qhh + ebh, 0.0)
                return cc
            lax.fori_loop(0, B, row, 0)

        def superstep(s, c):
            row_base = wid * rows_pt + s * KC * B
            idx_row = wid * (rows_pt // B) + s * KC
            pltpu.sync_copy(dst_hbm.at[pl.ds(idx_row, KC)], dbuf)
            pltpu.sync_copy(src_hbm.at[pl.ds(idx_row, KC)], sbuf)
            issue_gather(row_base, 0, sets[0])
            issue_gather(row_base, 1, sets[1])

            def pair(i, cc):
                for b in (0, 1):
                    ch = 2 * i + b
                    wait_gather(sets[b])

                    @pl.when(i > 0)
                    def _():
                        wait_scatter(sets[b])
                    compute(sets[b])
                    issue_scatter(ch, sets[b])

                    @pl.when(i < pairs - 1)
                    def _():
                        issue_gather(row_base, ch + 2, sets[b])
                return cc
            lax.fori_loop(0, pairs, pair, 0)
            wait_scatter(sets[0])
            wait_scatter(sets[1])
            return c
        lax.fori_loop(0, n_ss, superstep, 0)

        plsc.subcore_barrier()
        pltpu.sync_copy(h_sh.at[pl.ds(sid * rpt, rpt)],
                        h_out.at[cid, pl.ds(sid * rpt, rpt)])

    mesh = plsc.VectorSubcoreMesh(core_axis_name="c", subcore_axis_name="s")
    return pl.kernel(
        body,
        out_type=jax.ShapeDtypeStruct((NC, n_nodes, hidw), F32),
        mesh=mesh,
        compiler_params=pltpu.CompilerParams(use_tc_tiling_on_sc=False),
        scratch_types=[
            pltpu.VMEM_SHARED((n_nodes, hidw), F32),
            pltpu.VMEM((KC, B2), I32),
            pltpu.VMEM((KC, B2), I32),
            pltpu.VMEM((B2, hw), I32),
            pltpu.VMEM((B2, hw), I32),
            pltpu.VMEM((B, hid), I32),
            pltpu.VMEM((B2, hw), I32),
            pltpu.VMEM((B2, hw), I32),
            pltpu.VMEM((B, hid), I32),
            pltpu.VMEM((B2, hidw), F32),
            pltpu.VMEM((B2, hidw), F32),
            pltpu.SemaphoreType.DMA,
            pltpu.SemaphoreType.DMA,
            pltpu.SemaphoreType.DMA,
            pltpu.SemaphoreType.DMA,
            pltpu.SemaphoreType.DMA,
            pltpu.SemaphoreType.DMA,
            pltpu.SemaphoreType.DMA,
            pltpu.SemaphoreType.DMA,
        ],
    )


def kernel(node_features, edge_index, edge_features, mW1, mb1, mW2, mb2,
           uW1, ub1, uW2, ub2, gamma, beta):
    x = node_features
    n, hid = x.shape
    ne, edim = edge_features.shape
    src_flat = edge_index[0].astype(I32)
    dst_flat = edge_index[1].astype(I32)

    npk = ne // 2
    half = EBLK // 2
    nblk = ne // EBLK

    def remap(v):
        return v.reshape(nblk, 2, half // 20, 20).transpose(
            (0, 2, 1, 3)).reshape(npk // 20, 40)
    dstx = remap(dst_flat)
    srcx = remap(src_flat)

    wp = mW1[:hid]
    wq = mW1[hid:2 * hid]
    we = mW1[2 * hid:].astype(BF16)

    nb = 10
    bn = n // nb
    p, q = pl.pallas_call(
        _pq_body,
        grid=(nb,),
        in_specs=[
            pl.BlockSpec((bn, hid), lambda i: (i, 0)),
            pl.BlockSpec((hid, hid), lambda i: (0, 0)),
            pl.BlockSpec((hid, hid), lambda i: (0, 0)),
        ],
        out_specs=[
            pl.BlockSpec((bn, hid // 2), lambda i: (i, 0)),
            pl.BlockSpec((bn, hid // 2), lambda i: (i, 0)),
        ],
        out_shape=[
            jax.ShapeDtypeStruct((n, hid // 2), I32),
            jax.ShapeDtypeStruct((n, hid // 2), I32),
        ],
    )(x, wp, wq)

    e = pl.pallas_call(
        _edge_body,
        grid=(nblk,),
        in_specs=[
            pl.BlockSpec((EBLK, edim), lambda i: (i, 0)),
            pl.BlockSpec((edim, hid), lambda i: (0, 0)),
            pl.BlockSpec((1, hid), lambda i: (0, 0)),
        ],
        out_specs=pl.BlockSpec((half, hid), lambda i: (i, 0)),
        out_shape=jax.ShapeDtypeStruct((npk, hid), I32),
    )(edge_features, we, mb1.reshape(1, hid))

    hpart = _make_sc_edge_kernel(n, ne, hid)(p, q, e, dstx, srcx)

    out = pl.pallas_call(
        _post_body,
        grid=(nb,),
        in_specs=[
            pl.BlockSpec((bn, hid), lambda i: (i, 0)),
            pl.BlockSpec((NC, bn, hid + L), lambda i: (0, i, 0)),
            pl.BlockSpec((hid, hid), lambda i: (0, 0)),
            pl.BlockSpec((1, hid), lambda i: (0, 0)),
            pl.BlockSpec((hid, hid), lambda i: (0, 0)),
            pl.BlockSpec((hid, hid), lambda i: (0, 0)),
            pl.BlockSpec((1, hid), lambda i: (0, 0)),
            pl.BlockSpec((hid, hid), lambda i: (0, 0)),
            pl.BlockSpec((1, hid), lambda i: (0, 0)),
            pl.BlockSpec((1, hid), lambda i: (0, 0)),
            pl.BlockSpec((1, hid), lambda i: (0, 0)),
        ],
        out_specs=pl.BlockSpec((bn, hid), lambda i: (i, 0)),
        out_shape=jax.ShapeDtypeStruct((n, hid), F32),
    )(x, hpart, mW2, mb2.reshape(1, hid), uW1[:hid], uW1[hid:],
      ub1.reshape(1, hid), uW2, ub2.reshape(1, hid), gamma.reshape(1, hid),
      beta.reshape(1, hid))
    return out


# speedup vs baseline: 1.6246x; 1.0236x over previous
import functools

import jax
import jax.numpy as jnp
from jax import lax
from jax.experimental import pallas as pl
from jax.experimental.pallas import tpu as pltpu
from jax.experimental.pallas import tpu_sc as plsc

F32 = jnp.float32
BF16 = jnp.bfloat16
I32 = jnp.int32
NC = 2
NS = 16
L = 16

EBLK = 4000


def _bf16_bits(x):
    return lax.bitcast_convert_type(x.astype(BF16), jnp.uint16).astype(I32)


def _pq_body(x_ref, wp_ref, wq_ref, p_ref, q_ref):
    x = x_ref[...]
    h = x.shape[1] // 2

    def packc(m):
        return lax.shift_left(_bf16_bits(m[:, h:]), 16) | _bf16_bits(m[:, :h])
    p_ref[...] = packc(jnp.dot(x, wp_ref[...], preferred_element_type=F32))
    q_ref[...] = packc(jnp.dot(x, wq_ref[...], preferred_element_type=F32))


def _edge_body(ef_ref, we_ref, b_ref, e_ref):
    half = EBLK // 2
    ea = jnp.dot(ef_ref[...].astype(BF16), we_ref[...],
                 preferred_element_type=F32) + b_ref[...]
    e_ref[...] = lax.shift_left(_bf16_bits(ea[half:]), 16) \
        | _bf16_bits(ea[:half])


def _post_body(x_ref, hp_ref, dp_ref, w2_ref, b2_ref, wua_ref, wub_ref,
               bu1_ref, wu2_ref, bu2_ref, g_ref, be_ref, o_ref):
    x = x_ref[...]
    hsum = hp_ref[0] + hp_ref[1]
    deg = dp_ref[0][:, 0:1] + dp_ref[1][:, 0:1]
    agg = jnp.dot(hsum, w2_ref[...], preferred_element_type=F32) \
        + deg * b2_ref[...]
    u1 = jnp.maximum(
        jnp.dot(x, wua_ref[...], preferred_element_type=F32)
        + jnp.dot(agg, wub_ref[...], preferred_element_type=F32)
        + bu1_ref[...], 0.0)
    upd = jnp.dot(u1, wu2_ref[...], preferred_element_type=F32) + bu2_ref[...]
    y = x + upd
    mu = jnp.mean(y, axis=-1, keepdims=True)
    var = jnp.mean((y - mu) ** 2, axis=-1, keepdims=True)
    o_ref[...] = (y - mu) * lax.rsqrt(var + 1e-5) * g_ref[...] + be_ref[...]


def _make_sc_edge_kernel(n_nodes, n_edges, hid):
    nw = NC * NS
    npk = n_edges // 2
    rows_pt = npk // nw
    B = 20
    KC = 50
    n_ss = rows_pt // (KC * B)
    pairs = KC // 2
    rpt = n_nodes // NS
    B2 = 2 * B
    hw = hid // 2
    mask_hi = jnp.int32(-65536)

    def body(p_hbm, q_hbm, e_hbm, dst_hbm, src_hbm,
             h_out, d_out,
             h_sh, d_sh,
             dbuf, sbuf,
             pr0, qr0, er0, pr1, qr1, er1,
             hr0, hr1,
             ones_v, zd,
             sp0, sq0, se0, sp1, sq1, se1,
             shh0, shd0, shh1, shd1):
        cid = lax.axis_index("c")
        sid = lax.axis_index("s")
        wid = cid * NS + sid

        zero16 = jnp.zeros((L,), F32)
        ones16 = jnp.ones((L,), F32)

        def fill(i, c):
            for j in range(hid // L):
                hr0[i, pl.ds(j * L, L)] = zero16
            zd[i, pl.ds(0, L)] = zero16
            ones_v[i, pl.ds(0, L)] = ones16
            return c
        lax.fori_loop(0, B2, fill, 0)

        nfull = rpt // B2
        rem = rpt - nfull * B2
        for t in range(nfull):
            pltpu.sync_copy(hr0, h_sh.at[pl.ds(sid * rpt + t * B2, B2)])
            pltpu.sync_copy(zd, d_sh.at[pl.ds(sid * rpt + t * B2, B2)])
        if rem:
            pltpu.sync_copy(hr0.at[pl.ds(0, rem)],
                            h_sh.at[pl.ds(sid * rpt + nfull * B2, rem)])
            pltpu.sync_copy(zd.at[pl.ds(0, rem)],
                            d_sh.at[pl.ds(sid * rpt + nfull * B2, rem)])
        plsc.subcore_barrier()

        sets = ((pr0, qr0, er0, hr0, sp0, sq0, se0, shh0, shd0),
                (pr1, qr1, er1, hr1, sp1, sq1, se1, shh1, shd1))

        def issue_gather(row_base, c, bset):
            pr, qr, er, hr, sp, sq, se, shh, shd = bset
            pltpu.async_copy(p_hbm.at[dbuf.at[c]], pr, sp)
            pltpu.async_copy(q_hbm.at[sbuf.at[c]], qr, sq)
            pltpu.async_copy(e_hbm.at[pl.ds(row_base + c * B, B)], er, se)

        def wait_gather(bset):
            pr, qr, er, hr, sp, sq, se, shh, shd = bset
            pltpu.make_async_copy(p_hbm.at[pl.ds(0, B2)], pr, sp).wait()
            pltpu.make_async_copy(q_hbm.at[pl.ds(0, B2)], qr, sq).wait()
            pltpu.make_async_copy(e_hbm.at[pl.ds(0, B)], er, se).wait()

        def issue_scatter(c, bset):
            pr, qr, er, hr, sp, sq, se, shh, shd = bset
            pltpu.async_copy(hr, h_sh.at[dbuf.at[c]], shh, add=True)
            pltpu.async_copy(ones_v, d_sh.at[dbuf.at[c]], shd, add=True)

        def wait_scatter(bset):
            pr, qr, er, hr, sp, sq, se, shh, shd = bset
            pltpu.make_async_copy(hr, h_sh.at[pl.ds(0, B2)], shh).wait()
            pltpu.make_async_copy(ones_v, d_sh.at[pl.ds(0, B2)], shd).wait()

        def split(w):
            lo = lax.bitcast_convert_type(lax.shift_left(w, 16), F32)
            hi = lax.bitcast_convert_type(lax.bitwise_and(w, mask_hi), F32)
            return lo, hi

        def compute(bset):
            pr, qr, er, hr, sp, sq, se, shh, shd = bset

            def row(rr, cc):
                for t in range(4):
                    sl = pl.ds(t * L, L)
                    sh = pl.ds(hw + t * L, L)
                    pll, plh = split(pr[rr, sl])
                    phl, phh = split(pr[B + rr, sl])
                    qll, qlh = split(qr[rr, sl])
                    qhl, qhh = split(qr[B + rr, sl])
                    eal, eah = split(er[rr, sl])
                    ebl, ebh = split(er[rr, sh])
                    hr[rr, sl] = jnp.maximum(pll + qll + eal, 0.0)
                    hr[rr, sh] = jnp.maximum(plh + qlh + ebl, 0.0)
                    hr[B + rr, sl] = jnp.maximum(phl + qhl + eah, 0.0)
                    hr[B + rr, sh] = jnp.maximum(phh + qhh + ebh, 0.0)
                return cc
            lax.fori_loop(0, B, row, 0)

        def superstep(s, c):
            row_base = wid * rows_pt + s * KC * B
            idx_row = wid * (rows_pt // B) + s * KC
            pltpu.sync_copy(dst_hbm.at[pl.ds(idx_row, KC)], dbuf)
            pltpu.sync_copy(src_hbm.at[pl.ds(idx_row, KC)], sbuf)
            issue_gather(row_base, 0, sets[0])
            issue_gather(row_base, 1, sets[1])

            def pair(i, cc):
                for b in (0, 1):
                    ch = 2 * i + b
                    wait_gather(sets[b])

                    @pl.when(i > 0)
                    def _():
                        wait_scatter(sets[b])
                    compute(sets[b])
                    issue_scatter(ch, sets[b])

                    @pl.when(i < pairs - 1)
                    def _():
                        issue_gather(row_base, ch + 2, sets[b])
                return cc
            lax.fori_loop(0, pairs, pair, 0)
            wait_scatter(sets[0])
            wait_scatter(sets[1])
            return c
        lax.fori_loop(0, n_ss, superstep, 0)

        plsc.subcore_barrier()
        pltpu.sync_copy(h_sh.at[pl.ds(sid * rpt, rpt)],
                        h_out.at[cid, pl.ds(sid * rpt, rpt)])
        pltpu.sync_copy(d_sh.at[pl.ds(sid * rpt, rpt)],
                        d_out.at[cid, pl.ds(sid * rpt, rpt)])

    mesh = plsc.VectorSubcoreMesh(core_axis_name="c", subcore_axis_name="s")
    return pl.kernel(
        body,
        out_type=[
            jax.ShapeDtypeStruct((NC, n_nodes, hid), F32),
            jax.ShapeDtypeStruct((NC, n_nodes, L), F32),
        ],
        mesh=mesh,
        compiler_params=pltpu.CompilerParams(use_tc_tiling_on_sc=False),
        scratch_types=[
            pltpu.VMEM_SHARED((n_nodes, hid), F32),
            pltpu.VMEM_SHARED((n_nodes, L), F32),
            pltpu.VMEM((KC, B2), I32),
            pltpu.VMEM((KC, B2), I32),
            pltpu.VMEM((B2, hw), I32),
            pltpu.VMEM((B2, hw), I32),
            pltpu.VMEM((B, hid), I32),
            pltpu.VMEM((B2, hw), I32),
            pltpu.VMEM((B2, hw), I32),
            pltpu.VMEM((B, hid), I32),
            pltpu.VMEM((B2, hid), F32),
            pltpu.VMEM((B2, hid), F32),
            pltpu.VMEM((B2, L), F32),
            pltpu.VMEM((B2, L), F32),
            pltpu.SemaphoreType.DMA,
            pltpu.SemaphoreType.DMA,
            pltpu.SemaphoreType.DMA,
            pltpu.SemaphoreType.DMA,
            pltpu.SemaphoreType.DMA,
            pltpu.SemaphoreType.DMA,
            pltpu.SemaphoreType.DMA,
            pltpu.SemaphoreType.DMA,
            pltpu.SemaphoreType.DMA,
            pltpu.SemaphoreType.DMA,
        ],
    )


def kernel(node_features, edge_index, edge_features, mW1, mb1, mW2, mb2,
           uW1, ub1, uW2, ub2, gamma, beta):
    x = node_features
    n, hid = x.shape
    ne, edim = edge_features.shape
    src_flat = edge_index[0].astype(I32)
    dst_flat = edge_index[1].astype(I32)

    npk = ne // 2
    half = EBLK // 2
    nblk = ne // EBLK

    def remap(v):
        return v.reshape(nblk, 2, half // 20, 20).transpose(
            (0, 2, 1, 3)).reshape(npk // 20, 40)
    dstx = remap(dst_flat)
    srcx = remap(src_flat)

    wp = mW1[:hid]
    wq = mW1[hid:2 * hid]
    we = mW1[2 * hid:].astype(BF16)

    nb = 10
    bn = n // nb
    p, q = pl.pallas_call(
        _pq_body,
        grid=(nb,),
        in_specs=[
            pl.BlockSpec((bn, hid), lambda i: (i, 0)),
            pl.BlockSpec((hid, hid), lambda i: (0, 0)),
            pl.BlockSpec((hid, hid), lambda i: (0, 0)),
        ],
        out_specs=[
            pl.BlockSpec((bn, hid // 2), lambda i: (i, 0)),
            pl.BlockSpec((bn, hid // 2), lambda i: (i, 0)),
        ],
        out_shape=[
            jax.ShapeDtypeStruct((n, hid // 2), I32),
            jax.ShapeDtypeStruct((n, hid // 2), I32),
        ],
    )(x, wp, wq)

    e = pl.pallas_call(
        _edge_body,
        grid=(nblk,),
        in_specs=[
            pl.BlockSpec((EBLK, edim), lambda i: (i, 0)),
            pl.BlockSpec((edim, hid), lambda i: (0, 0)),
            pl.BlockSpec((1, hid), lambda i: (0, 0)),
        ],
        out_specs=pl.BlockSpec((half, hid), lambda i: (i, 0)),
        out_shape=jax.ShapeDtypeStruct((npk, hid), I32),
    )(edge_features, we, mb1.reshape(1, hid))

    hpart, dpart = _make_sc_edge_kernel(n, ne, hid)(p, q, e, dstx, srcx)

    out = pl.pallas_call(
        _post_body,
        grid=(nb,),
        in_specs=[
            pl.BlockSpec((bn, hid), lambda i: (i, 0)),
            pl.BlockSpec((NC, bn, hid), lambda i: (0, i, 0)),
            pl.BlockSpec((NC, bn, L), lambda i: (0, i, 0)),
            pl.BlockSpec((hid, hid), lambda i: (0, 0)),
            pl.BlockSpec((1, hid), lambda i: (0, 0)),
            pl.BlockSpec((hid, hid), lambda i: (0, 0)),
            pl.BlockSpec((hid, hid), lambda i: (0, 0)),
            pl.BlockSpec((1, hid), lambda i: (0, 0)),
            pl.BlockSpec((hid, hid), lambda i: (0, 0)),
            pl.BlockSpec((1, hid), lambda i: (0, 0)),
            pl.BlockSpec((1, hid), lambda i: (0, 0)),
            pl.BlockSpec((1, hid), lambda i: (0, 0)),
        ],
        out_specs=pl.BlockSpec((bn, hid), lambda i: (i, 0)),
        out_shape=jax.ShapeDtypeStruct((n, hid), F32),
    )(x, hpart, dpart, mW2, mb2.reshape(1, hid), uW1[:hid], uW1[hid:],
      ub1.reshape(1, hid), uW2, ub2.reshape(1, hid), gamma.reshape(1, hid),
      beta.reshape(1, hid))
    return out


# speedup vs baseline: 1.6446x; 1.0123x over previous
import functools

import jax
import jax.numpy as jnp
from jax import lax
from jax.experimental import pallas as pl
from jax.experimental.pallas import tpu as pltpu
from jax.experimental.pallas import tpu_sc as plsc

F32 = jnp.float32
BF16 = jnp.bfloat16
I32 = jnp.int32
NC = 2
NS = 16
L = 16

EBLK = 8000


def _bf16_bits(x):
    return lax.bitcast_convert_type(x.astype(BF16), jnp.uint16).astype(I32)


def _pq_body(x_ref, wp_ref, wq_ref, p_ref, q_ref):
    x = x_ref[...]
    h = x.shape[1] // 2

    def packc(m):
        return lax.shift_left(_bf16_bits(m[:, h:]), 16) | _bf16_bits(m[:, :h])
    p_ref[...] = packc(jnp.dot(x, wp_ref[...], preferred_element_type=F32))
    q_ref[...] = packc(jnp.dot(x, wq_ref[...], preferred_element_type=F32))


def _edge_body(ef_ref, we_ref, b_ref, e_ref):
    half = EBLK // 2
    ea = jnp.dot(ef_ref[...].astype(BF16), we_ref[...],
                 preferred_element_type=F32) + b_ref[...]
    e_ref[...] = lax.shift_left(_bf16_bits(ea[half:]), 16) \
        | _bf16_bits(ea[:half])


def _post_body(x_ref, hp_ref, dp_ref, w2_ref, b2_ref, wua_ref, wub_ref,
               bu1_ref, wu2_ref, bu2_ref, g_ref, be_ref, o_ref):
    x = x_ref[...]
    hsum = hp_ref[0] + hp_ref[1]
    deg = dp_ref[0][:, 0:1] + dp_ref[1][:, 0:1]
    agg = jnp.dot(hsum, w2_ref[...], preferred_element_type=F32) \
        + deg * b2_ref[...]
    u1 = jnp.maximum(
        jnp.dot(x, wua_ref[...], preferred_element_type=F32)
        + jnp.dot(agg, wub_ref[...], preferred_element_type=F32)
        + bu1_ref[...], 0.0)
    upd = jnp.dot(u1, wu2_ref[...], preferred_element_type=F32) + bu2_ref[...]
    y = x + upd
    mu = jnp.mean(y, axis=-1, keepdims=True)
    var = jnp.mean((y - mu) ** 2, axis=-1, keepdims=True)
    o_ref[...] = (y - mu) * lax.rsqrt(var + 1e-5) * g_ref[...] + be_ref[...]


def _make_sc_edge_kernel(n_nodes, n_edges, hid):
    nw = NC * NS
    npk = n_edges // 2
    rows_pt = npk // nw
    B = 20
    KC = 50
    n_ss = rows_pt // (KC * B)
    pairs = KC // 2
    rpt = n_nodes // NS
    B2 = 2 * B
    hw = hid // 2
    mask_hi = jnp.int32(-65536)

    def body(p_hbm, q_hbm, e_hbm, dst_hbm, src_hbm,
             h_out, d_out,
             h_sh, d_sh,
             dbuf, sbuf,
             pr0, qr0, er0, pr1, qr1, er1,
             hr0, hr1,
             ones_v, zd,
             sp0, sq0, se0, sp1, sq1, se1,
             shh0, shd0, shh1, shd1):
        cid = lax.axis_index("c")
        sid = lax.axis_index("s")
        wid = cid * NS + sid

        zero16 = jnp.zeros((L,), F32)
        ones16 = jnp.ones((L,), F32)

        def fill(i, c):
            for j in range(hid // L):
                hr0[i, pl.ds(j * L, L)] = zero16
            zd[i, pl.ds(0, L)] = zero16
            ones_v[i, pl.ds(0, L)] = ones16
            return c
        lax.fori_loop(0, B2, fill, 0)

        nfull = rpt // B2
        rem = rpt - nfull * B2
        for t in range(nfull):
            pltpu.sync_copy(hr0, h_sh.at[pl.ds(sid * rpt + t * B2, B2)])
            pltpu.sync_copy(zd, d_sh.at[pl.ds(sid * rpt + t * B2, B2)])
        if rem:
            pltpu.sync_copy(hr0.at[pl.ds(0, rem)],
                            h_sh.at[pl.ds(sid * rpt + nfull * B2, rem)])
            pltpu.sync_copy(zd.at[pl.ds(0, rem)],
                            d_sh.at[pl.ds(sid * rpt + nfull * B2, rem)])
        plsc.subcore_barrier()

        sets = ((pr0, qr0, er0, hr0, sp0, sq0, se0, shh0, shd0),
                (pr1, qr1, er1, hr1, sp1, sq1, se1, shh1, shd1))

        def issue_gather(row_base, c, bset):
            pr, qr, er, hr, sp, sq, se, shh, shd = bset
            pltpu.async_copy(p_hbm.at[dbuf.at[c]], pr, sp)
            pltpu.async_copy(q_hbm.at[sbuf.at[c]], qr, sq)
            pltpu.async_copy(e_hbm.at[pl.ds(row_base + c * B, B)], er, se)

        def wait_gather(bset):
            pr, qr, er, hr, sp, sq, se, shh, shd = bset
            pltpu.make_async_copy(p_hbm.at[pl.ds(0, B2)], pr, sp).wait()
            pltpu.make_async_copy(q_hbm.at[pl.ds(0, B2)], qr, sq).wait()
            pltpu.make_async_copy(e_hbm.at[pl.ds(0, B)], er, se).wait()

        def issue_scatter(c, bset):
            pr, qr, er, hr, sp, sq, se, shh, shd = bset
            pltpu.async_copy(hr, h_sh.at[dbuf.at[c]], shh, add=True)
            pltpu.async_copy(ones_v, d_sh.at[dbuf.at[c]], shd, add=True)

        def wait_scatter(bset):
            pr, qr, er, hr, sp, sq, se, shh, shd = bset
            pltpu.make_async_copy(hr, h_sh.at[pl.ds(0, B2)], shh).wait()
            pltpu.make_async_copy(ones_v, d_sh.at[pl.ds(0, B2)], shd).wait()

        def split(w):
            lo = lax.bitcast_convert_type(lax.shift_left(w, 16), F32)
            hi = lax.bitcast_convert_type(lax.bitwise_and(w, mask_hi), F32)
            return lo, hi

        def compute(bset):
            pr, qr, er, hr, sp, sq, se, shh, shd = bset

            def row(rr, cc):
                for t in range(4):
                    sl = pl.ds(t * L, L)
                    sh = pl.ds(hw + t * L, L)
                    pll, plh = split(pr[rr, sl])
                    phl, phh = split(pr[B + rr, sl])
                    qll, qlh = split(qr[rr, sl])
                    qhl, qhh = split(qr[B + rr, sl])
                    eal, eah = split(er[rr, sl])
                    ebl, ebh = split(er[rr, sh])
                    hr[rr, sl] = jnp.maximum(pll + qll + eal, 0.0)
                    hr[rr, sh] = jnp.maximum(plh + qlh + ebl, 0.0)
                    hr[B + rr, sl] = jnp.maximum(phl + qhl + eah, 0.0)
                    hr[B + rr, sh] = jnp.maximum(phh + qhh + ebh, 0.0)
                return cc
            lax.fori_loop(0, B, row, 0)

        def superstep(s, c):
            row_base = wid * rows_pt + s * KC * B
            idx_row = wid * (rows_pt // B) + s * KC
            pltpu.sync_copy(dst_hbm.at[pl.ds(idx_row, KC)], dbuf)
            pltpu.sync_copy(src_hbm.at[pl.ds(idx_row, KC)], sbuf)
            issue_gather(row_base, 0, sets[0])
            issue_gather(row_base, 1, sets[1])

            def pair(i, cc):
                for b in (0, 1):
                    ch = 2 * i + b
                    wait_gather(sets[b])

                    @pl.when(i > 0)
                    def _():
                        wait_scatter(sets[b])
                    compute(sets[b])
                    issue_scatter(ch, sets[b])

                    @pl.when(i < pairs - 1)
                    def _():
                        issue_gather(row_base, ch + 2, sets[b])
                return cc
            lax.fori_loop(0, pairs, pair, 0)
            wait_scatter(sets[0])
            wait_scatter(sets[1])
            return c
        lax.fori_loop(0, n_ss, superstep, 0)

        plsc.subcore_barrier()
        pltpu.sync_copy(h_sh.at[pl.ds(sid * rpt, rpt)],
                        h_out.at[cid, pl.ds(sid * rpt, rpt)])
        pltpu.sync_copy(d_sh.at[pl.ds(sid * rpt, rpt)],
                        d_out.at[cid, pl.ds(sid * rpt, rpt)])

    mesh = plsc.VectorSubcoreMesh(core_axis_name="c", subcore_axis_name="s")
    return pl.kernel(
        body,
        out_type=[
            jax.ShapeDtypeStruct((NC, n_nodes, hid), F32),
            jax.ShapeDtypeStruct((NC, n_nodes, L), F32),
        ],
        mesh=mesh,
        compiler_params=pltpu.CompilerParams(use_tc_tiling_on_sc=False),
        scratch_types=[
            pltpu.VMEM_SHARED((n_nodes, hid), F32),
            pltpu.VMEM_SHARED((n_nodes, L), F32),
            pltpu.VMEM((KC, B2), I32),
            pltpu.VMEM((KC, B2), I32),
            pltpu.VMEM((B2, hw), I32),
            pltpu.VMEM((B2, hw), I32),
            pltpu.VMEM((B, hid), I32),
            pltpu.VMEM((B2, hw), I32),
            pltpu.VMEM((B2, hw), I32),
            pltpu.VMEM((B, hid), I32),
            pltpu.VMEM((B2, hid), F32),
            pltpu.VMEM((B2, hid), F32),
            pltpu.VMEM((B2, L), F32),
            pltpu.VMEM((B2, L), F32),
            pltpu.SemaphoreType.DMA,
            pltpu.SemaphoreType.DMA,
            pltpu.SemaphoreType.DMA,
            pltpu.SemaphoreType.DMA,
            pltpu.SemaphoreType.DMA,
            pltpu.SemaphoreType.DMA,
            pltpu.SemaphoreType.DMA,
            pltpu.SemaphoreType.DMA,
            pltpu.SemaphoreType.DMA,
            pltpu.SemaphoreType.DMA,
        ],
    )


def kernel(node_features, edge_index, edge_features, mW1, mb1, mW2, mb2,
           uW1, ub1, uW2, ub2, gamma, beta):
    x = node_features
    n, hid = x.shape
    ne, edim = edge_features.shape
    src_flat = edge_index[0].astype(I32)
    dst_flat = edge_index[1].astype(I32)

    npk = ne // 2
    half = EBLK // 2
    nblk = ne // EBLK

    def remap(v):
        return v.reshape(nblk, 2, half // 20, 20).transpose(
            (0, 2, 1, 3)).reshape(npk // 20, 40)
    dstx = remap(dst_flat)
    srcx = remap(src_flat)

    wp = mW1[:hid]
    wq = mW1[hid:2 * hid]
    we = mW1[2 * hid:].astype(BF16)

    nb = 10
    bn = n // nb
    p, q = pl.pallas_call(
        _pq_body,
        grid=(nb,),
        in_specs=[
            pl.BlockSpec((bn, hid), lambda i: (i, 0)),
            pl.BlockSpec((hid, hid), lambda i: (0, 0)),
            pl.BlockSpec((hid, hid), lambda i: (0, 0)),
        ],
        out_specs=[
            pl.BlockSpec((bn, hid // 2), lambda i: (i, 0)),
            pl.BlockSpec((bn, hid // 2), lambda i: (i, 0)),
        ],
        out_shape=[
            jax.ShapeDtypeStruct((n, hid // 2), I32),
            jax.ShapeDtypeStruct((n, hid // 2), I32),
        ],
    )(x, wp, wq)

    e = pl.pallas_call(
        _edge_body,
        grid=(nblk,),
        in_specs=[
            pl.BlockSpec((EBLK, edim), lambda i: (i, 0)),
            pl.BlockSpec((edim, hid), lambda i: (0, 0)),
            pl.BlockSpec((1, hid), lambda i: (0, 0)),
        ],
        out_specs=pl.BlockSpec((half, hid), lambda i: (i, 0)),
        out_shape=jax.ShapeDtypeStruct((npk, hid), I32),
    )(edge_features, we, mb1.reshape(1, hid))

    hpart, dpart = _make_sc_edge_kernel(n, ne, hid)(p, q, e, dstx, srcx)

    out = pl.pallas_call(
        _post_body,
        grid=(nb,),
        in_specs=[
            pl.BlockSpec((bn, hid), lambda i: (i, 0)),
            pl.BlockSpec((NC, bn, hid), lambda i: (0, i, 0)),
            pl.BlockSpec((NC, bn, L), lambda i: (0, i, 0)),
            pl.BlockSpec((hid, hid), lambda i: (0, 0)),
            pl.BlockSpec((1, hid), lambda i: (0, 0)),
            pl.BlockSpec((hid, hid), lambda i: (0, 0)),
            pl.BlockSpec((hid, hid), lambda i: (0, 0)),
            pl.BlockSpec((1, hid), lambda i: (0, 0)),
            pl.BlockSpec((hid, hid), lambda i: (0, 0)),
            pl.BlockSpec((1, hid), lambda i: (0, 0)),
            pl.BlockSpec((1, hid), lambda i: (0, 0)),
            pl.BlockSpec((1, hid), lambda i: (0, 0)),
        ],
        out_specs=pl.BlockSpec((bn, hid), lambda i: (i, 0)),
        out_shape=jax.ShapeDtypeStruct((n, hid), F32),
    )(x, hpart, dpart, mW2, mb2.reshape(1, hid), uW1[:hid], uW1[hid:],
      ub1.reshape(1, hid), uW2, ub2.reshape(1, hid), gamma.reshape(1, hid),
      beta.reshape(1, hid))
    return out
